# baseline (speedup 1.0000x reference)
import functools

import jax
import jax.numpy as jnp
from jax import lax
from jax.experimental import pallas as pl
from jax.experimental.pallas import tpu as pltpu
from jax.experimental.pallas import tpu_sc as plsc

NUM_CORES = 2
NUM_SUBCORES = 16
NUM_WORKERS = NUM_CORES * NUM_SUBCORES
BATCH = 16384
DIM = 64
ROWS_PER_WORKER = BATCH // NUM_WORKERS
CHUNK = 128
CHUNKS_PER_WORKER = ROWS_PER_WORKER // CHUNK


def _sc_gather_body(uidx_hbm, midx_hbm, umf_hbm, mmf_hbm, umlp_hbm, mmlp_hbm,
                    umf_out, mmf_out, umlp_out, mmlp_out,
                    idx_u, idx_m, rows_a, rows_b, sem):
    wid = lax.axis_index("s") * NUM_CORES + lax.axis_index("c")
    base = wid * ROWS_PER_WORKER
    cbase = wid * CHUNKS_PER_WORKER

    pltpu.sync_copy(uidx_hbm.at[pl.ds(cbase, CHUNKS_PER_WORKER)], idx_u)
    pltpu.sync_copy(midx_hbm.at[pl.ds(cbase, CHUNKS_PER_WORKER)], idx_m)

    def gather_pair(table_u, table_m):
        copies = []
        for k in range(CHUNKS_PER_WORKER):
            dst = pl.ds(k * CHUNK, CHUNK)
            copies.append(pltpu.async_copy(table_u.at[idx_u.at[k]], rows_a.at[dst], sem))
            copies.append(pltpu.async_copy(table_m.at[idx_m.at[k]], rows_b.at[dst], sem))
        for c in copies:
            c.wait()

    gather_pair(umf_hbm, mmf_hbm)
    pltpu.sync_copy(rows_a, umf_out.at[pl.ds(base, ROWS_PER_WORKER)])
    pltpu.sync_copy(rows_b, mmf_out.at[pl.ds(base, ROWS_PER_WORKER)])

    gather_pair(umlp_hbm, mmlp_hbm)
    pltpu.sync_copy(rows_a, umlp_out.at[pl.ds(base, ROWS_PER_WORKER)])
    pltpu.sync_copy(rows_b, mmlp_out.at[pl.ds(base, ROWS_PER_WORKER)])


_sc_gather = functools.partial(
    pl.kernel,
    mesh=plsc.VectorSubcoreMesh(core_axis_name="c", subcore_axis_name="s"),
    out_type=[jax.ShapeDtypeStruct((BATCH, DIM), jnp.float32)] * 4,
    scratch_types=[
        pltpu.VMEM((CHUNKS_PER_WORKER, CHUNK), jnp.int32),
        pltpu.VMEM((CHUNKS_PER_WORKER, CHUNK), jnp.int32),
        pltpu.VMEM((ROWS_PER_WORKER, DIM), jnp.float32),
        pltpu.VMEM((ROWS_PER_WORKER, DIM), jnp.float32),
        pltpu.SemaphoreType.DMA,
    ],
    compiler_params=pltpu.CompilerParams(use_tc_tiling_on_sc=False),
)(_sc_gather_body)


TC_BLOCK = 2048


def _tc_dense_body(umf_ref, mmf_ref, umlp_ref, mmlp_ref,
                   w1a_ref, w1b_ref, b1_ref, wf0_ref, wf1_ref, bf_ref, out_ref):
    h = jnp.dot(umlp_ref[...], w1a_ref[...], preferred_element_type=jnp.float32)
    h = h + jnp.dot(mmlp_ref[...], w1b_ref[...], preferred_element_type=jnp.float32)
    h = jnp.maximum(h + b1_ref[...], 0.0)
    gmf = umf_ref[...] * mmf_ref[...]
    logit = jnp.sum(gmf * wf0_ref[...], axis=1, keepdims=True)
    logit = logit + jnp.sum(h * wf1_ref[...], axis=1, keepdims=True)
    logit = logit + bf_ref[0, 0]
    out_ref[...] = jax.nn.sigmoid(logit)


def _tc_dense(umf, mmf, umlp, mmlp, w1a, w1b, b1, wf0, wf1, bf):
    grid = BATCH // TC_BLOCK
    row_spec = pl.BlockSpec((TC_BLOCK, DIM), lambda i: (i, 0))
    full = pl.BlockSpec(lambda i: (0, 0))
    return pl.pallas_call(
        _tc_dense_body,
        grid=(grid,),
        in_specs=[row_spec, row_spec, row_spec, row_spec,
                  pl.BlockSpec((DIM, DIM), lambda i: (0, 0)),
                  pl.BlockSpec((DIM, DIM), lambda i: (0, 0)),
                  pl.BlockSpec((1, DIM), lambda i: (0, 0)),
                  pl.BlockSpec((1, DIM), lambda i: (0, 0)),
                  pl.BlockSpec((1, DIM), lambda i: (0, 0)),
                  pl.BlockSpec((1, 1), lambda i: (0, 0))],
        out_specs=pl.BlockSpec((TC_BLOCK, 1), lambda i: (i, 0)),
        out_shape=jax.ShapeDtypeStruct((BATCH, 1), jnp.float32),
    )(umf, mmf, umlp, mmlp, w1a, w1b, b1, wf0, wf1, bf)


def kernel(x, user_mf, movie_mf, user_mlp, movie_mlp, W1, b1, Wf, bf):
    u_idx = x[:, 0].reshape(BATCH // CHUNK, CHUNK)
    m_idx = x[:, 1].reshape(BATCH // CHUNK, CHUNK)
    umf_rows, mmf_rows, umlp_rows, mmlp_rows = _sc_gather(
        u_idx, m_idx, user_mf, movie_mf, user_mlp, movie_mlp)
    w1a = W1[:DIM]
    w1b = W1[DIM:]
    wf0 = Wf[:DIM].reshape(1, DIM)
    wf1 = Wf[DIM:].reshape(1, DIM)
    return _tc_dense(umf_rows, mmf_rows, umlp_rows, mmlp_rows,
                     w1a, w1b, b1.reshape(1, DIM), wf0, wf1, bf.reshape(1, 1))
